# Initial kernel scaffold; baseline (speedup 1.0000x reference)
import jax
import jax.numpy as jnp
from jax.experimental import pallas as pl


def kernel(pred):
    raise NotImplementedError("write your pallas kernel here")


# speedup vs baseline: 2.2597x; 2.2597x over previous
import jax
import jax.numpy as jnp
from jax import lax
from jax.experimental import pallas as pl
from jax.experimental.pallas import tpu as pltpu

_N = 1024
_GRID = 32
_CLS = 20
_OBJ_T = 0.1
_NMS_T = 0.5
_CHUNK = 256


def _decode_cols(p):
    c = lax.broadcasted_iota(jnp.int32, (_N, 1), 0)
    contain = p[:, 4:5]
    cls = p[:, 5:25]
    maxv = jnp.max(cls, axis=1, keepdims=True)
    ii = lax.broadcasted_iota(jnp.int32, (_N, _CLS), 1)
    clsi = jnp.min(jnp.where(cls == maxv, ii, _CLS), axis=1,
                   keepdims=True).astype(jnp.float32)
    pm = contain * maxv
    mask = (pm > _OBJ_T).astype(jnp.float32)
    xo = (c // _GRID).astype(jnp.float32)
    yo = (c % _GRID).astype(jnp.float32)
    bx, by = p[:, 0:1], p[:, 1:2]
    bw, bh = p[:, 2:3], p[:, 3:4]
    cx = (bx + xo) / _GRID
    cy = (by + yo) / _GRID
    return {
        "x1": (cx - 0.5 * bw) * mask,
        "y1": (cy - 0.5 * bh) * mask,
        "x2": (cx + 0.5 * bw) * mask,
        "y2": (cy + 0.5 * bh) * mask,
        "cls": clsi * mask,
        "pm": pm * mask,
    }


def _decode_rows(pt):
    c = lax.broadcasted_iota(jnp.int32, (1, _N), 1)
    contain = pt[4:5, :]
    cls = pt[5:25, :]
    maxv = jnp.max(cls, axis=0, keepdims=True)
    ii = lax.broadcasted_iota(jnp.int32, (_CLS, _N), 0)
    clsi = jnp.min(jnp.where(cls == maxv, ii, _CLS), axis=0,
                   keepdims=True).astype(jnp.float32)
    pm = contain * maxv
    mask = (pm > _OBJ_T).astype(jnp.float32)
    xo = (c // _GRID).astype(jnp.float32)
    yo = (c % _GRID).astype(jnp.float32)
    bx, by = pt[0:1, :], pt[1:2, :]
    bw, bh = pt[2:3, :], pt[3:4, :]
    cx = (bx + xo) / _GRID
    cy = (by + yo) / _GRID
    return {
        "x1": (cx - 0.5 * bw) * mask,
        "y1": (cy - 0.5 * bh) * mask,
        "x2": (cx + 0.5 * bw) * mask,
        "y2": (cy + 0.5 * bh) * mask,
        "cls": clsi * mask,
        "pm": pm * mask,
    }


def _body(pred_ref, predt_ref, out_ref, iou_scr):
    p = pred_ref[0]
    pt = predt_ref[0]

    col = _decode_cols(p)
    row = _decode_rows(pt)

    j_row = lax.broadcasted_iota(jnp.int32, (1, _N), 1)
    rank_chunks = []
    colsum = jnp.zeros((1, _N), jnp.int32)
    for rb in range(0, _N, _CHUNK):
        pm_c = col["pm"][rb:rb + _CHUNK, :]
        i_col = lax.broadcasted_iota(jnp.int32, (_CHUNK, 1), 0) + rb
        gt = row["pm"] > pm_c
        eq = (row["pm"] == pm_c) & (j_row < i_col)
        m = (gt | eq).astype(jnp.int32)
        rank_chunks.append(jnp.sum(m, axis=1, keepdims=True))
        colsum = colsum + jnp.sum(m, axis=0, keepdims=True)
    rank_c = jnp.concatenate(rank_chunks, axis=0)
    rank_r = (_N - 1) - colsum

    k_col = lax.broadcasted_iota(jnp.int32, (_N, 1), 0)
    k_row = j_row
    b8 = jnp.concatenate(
        [col["x1"], col["y1"], col["x2"], col["y2"], col["cls"], col["pm"],
         jnp.zeros((_N, 2), jnp.float32)], axis=1)
    perm = (rank_r == k_col).astype(jnp.float32)
    s = jax.lax.dot_general(perm, b8, (((1,), (0,)), ((), ())),
                            precision=jax.lax.Precision.HIGHEST,
                            preferred_element_type=jnp.float32)
    r8 = jnp.concatenate(
        [row["x1"], row["y1"], row["x2"], row["y2"], row["cls"], row["pm"],
         jnp.zeros((2, _N), jnp.float32)], axis=0)
    permt = (rank_c == k_row).astype(jnp.float32)
    sr = jax.lax.dot_general(r8, permt, (((1,), (0,)), ((), ())),
                             precision=jax.lax.Precision.HIGHEST,
                             preferred_element_type=jnp.float32)

    x1c, y1c, x2c, y2c = s[:, 0:1], s[:, 1:2], s[:, 2:3], s[:, 3:4]
    x1r, y1r, x2r, y2r = sr[0:1, :], sr[1:2, :], sr[2:3, :], sr[3:4, :]
    area_c = (x2c - x1c) * (y2c - y1c)
    area_r = (x2r - x1r) * (y2r - y1r)
    for rb in range(0, _N, _CHUNK):
        sl = slice(rb, rb + _CHUNK)
        xx1 = jnp.maximum(x1c[sl, :], x1r)
        yy1 = jnp.maximum(y1c[sl, :], y1r)
        xx2 = jnp.minimum(x2c[sl, :], x2r)
        yy2 = jnp.minimum(y2c[sl, :], y2r)
        iw = jnp.clip(xx2 - xx1, 0.0, None)
        ih = jnp.clip(yy2 - yy1, 0.0, None)
        inter = iw * ih
        union = area_c[sl, :] + area_r - inter
        iou_scr[sl, :] = inter / union

    valid = (sr[5:6, :] > _OBJ_T).astype(jnp.float32)

    def step(i, keep):
        iou_row = iou_scr[pl.ds(i, 1), :]
        cur = jnp.sum(jnp.where(j_row == i, keep, 0.0), axis=1, keepdims=True)
        sup = (iou_row > _NMS_T) & (j_row > i)
        return keep * (1.0 - jnp.where(sup, 1.0, 0.0) * cur)

    keep = lax.fori_loop(0, _N, step, valid)

    kcol_chunks = []
    for rb in range(0, _N, _CHUNK):
        i_col = lax.broadcasted_iota(jnp.int32, (_CHUNK, 1), 0) + rb
        sel = jnp.where(j_row == i_col, keep, 0.0)
        kcol_chunks.append(jnp.sum(sel, axis=1, keepdims=True))
    kcol = jnp.concatenate(kcol_chunks, axis=0)

    out_ref[0] = s[:, 0:6] * kcol


def kernel(pred):
    b = pred.shape[0]
    predr = pred.reshape(b, _N, 25)
    predt = jnp.swapaxes(predr, 1, 2)
    return pl.pallas_call(
        _body,
        grid=(b,),
        in_specs=[
            pl.BlockSpec((1, _N, 25), lambda i: (i, 0, 0)),
            pl.BlockSpec((1, 25, _N), lambda i: (i, 0, 0)),
        ],
        out_specs=pl.BlockSpec((1, _N, 6), lambda i: (i, 0, 0)),
        out_shape=jax.ShapeDtypeStruct((b, _N, 6), jnp.float32),
        scratch_shapes=[pltpu.VMEM((_N, _N), jnp.float32)],
    )(predr, predt)
